# Initial kernel scaffold; baseline (speedup 1.0000x reference)
import jax
import jax.numpy as jnp
from jax.experimental import pallas as pl


def kernel(x, edge_index, W1, b1, W2, b2, W3, b3, fW1, fb1, fW2, fb2):
    raise NotImplementedError("write your pallas kernel here")


# speedup vs baseline: 9.8192x; 9.8192x over previous
import functools

import jax
import jax.numpy as jnp
from jax import lax
from jax.experimental import pallas as pl
from jax.experimental.pallas import tpu as pltpu
from jax.experimental.pallas import tpu_sc as plsc

N = 50000
NP = 50176
E = 800000
EP = 802816
NBLK = 512
GRID = NP // NBLK
RPT = NP // 16
B = 128
KI = 4
ZR = RPT // 4

_mesh = plsc.VectorSubcoreMesh(core_axis_name="c", subcore_axis_name="s")


def _zero_fill(ref, rows, width):
    zero16 = jnp.zeros((16,), jnp.float32)
    per_row = width // 16

    def body(i, _):
        r = i // per_row
        c = i % per_row
        ref[r, pl.ds(c * 16, 16)] = zero16
        return 0

    lax.fori_loop(0, rows * per_row, body, 0)


def _make_sc_agg(C, D, gather):
    partial = C == 1
    cpc = 1 if partial else C // 2
    ne_tile = EP // 32 if partial else EP // 16
    outer = ne_tile // (KI * B)

    def body(*refs):
        if gather:
            src2, dst2, g_h, out_h, agg, src_i, dst_i, gbuf, zbuf, sem = refs
        else:
            dst2, out_h, agg, src_i, dst_i, gbuf, zbuf, sem = refs
        cid = lax.axis_index("c")
        sid = lax.axis_index("s")
        row0 = sid * RPT

        _zero_fill(zbuf, ZR, D)
        if not gather:
            one16 = jnp.ones((16,), jnp.float32)

            def ones_body(i, _):
                r = i // (D // 16)
                c = i % (D // 16)
                gbuf[r, pl.ds(c * 16, 16)] = one16
                return 0

            lax.fori_loop(0, KI * B * (D // 16), ones_body, 0)

        for r in range(cpc):
            chunk = cid * cpc + r
            base_row = chunk * NP
            if partial:
                er0 = (cid * 16 + sid) * (ne_tile // B)
            else:
                er0 = sid * (ne_tile // B)

            for z in range(4):
                pltpu.sync_copy(zbuf, agg.at[pl.ds(row0 + z * ZR, ZR), :])
            plsc.subcore_barrier()

            def outer_body(o, _):
                eb = er0 + o * KI
                pltpu.sync_copy(dst2.at[pl.ds(eb, KI), :], dst_i)
                if gather:
                    pltpu.sync_copy(src2.at[pl.ds(eb, KI), :], src_i)
                    if not partial:
                        def adj(k, _):
                            j = k // (B // 16)
                            kk = k % (B // 16)
                            v = src_i[j, pl.ds(kk * 16, 16)]
                            src_i[j, pl.ds(kk * 16, 16)] = v + base_row
                            return 0

                        lax.fori_loop(0, KI * (B // 16), adj, 0)
                    descs = [
                        pltpu.async_copy(
                            g_h.at[src_i.at[j]],
                            gbuf.at[pl.ds(j * B, B), :],
                            sem,
                        )
                        for j in range(KI)
                    ]
                    for d in descs:
                        d.wait()
                for j in range(KI):
                    pltpu.sync_copy(
                        gbuf.at[pl.ds(j * B, B), :],
                        agg.at[dst_i.at[j]],
                        add=True,
                    )
                return 0

            lax.fori_loop(0, outer, outer_body, 0)
            plsc.subcore_barrier()

            out_base = (cid if partial else chunk) * NP
            pltpu.sync_copy(
                agg.at[pl.ds(row0, RPT), :],
                out_h.at[pl.ds(out_base + row0, RPT), :],
            )

    n_out = (2 if partial else C) * NP
    return functools.partial(
        pl.kernel,
        out_type=jax.ShapeDtypeStruct((n_out, D), jnp.float32),
        mesh=_mesh,
        scratch_types=[
            pltpu.VMEM_SHARED((NP, D), jnp.float32),
            pltpu.VMEM((KI, B), jnp.int32),
            pltpu.VMEM((KI, B), jnp.int32),
            pltpu.VMEM((KI * B, D), jnp.float32),
            pltpu.VMEM((ZR, D), jnp.float32),
            pltpu.SemaphoreType.DMA,
        ],
        compiler_params=pltpu.CompilerParams(use_tc_tiling_on_sc=False),
    )(body)


_sc_deg = _make_sc_agg(1, 16, gather=False)
_sc_agg1 = _make_sc_agg(1, 16, gather=True)
_sc_agg4 = _make_sc_agg(4, 16, gather=True)
_sc_agg8 = _make_sc_agg(8, 16, gather=True)


def _tc_prep(degp, x_pad):
    def body(degp_ref, x_ref, dinv_ref, g0_ref):
        deg = degp_ref[0, :, 0:1] + degp_ref[1, :, 0:1] + 1.0
        dv = lax.rsqrt(deg)
        dinv_ref[...] = jnp.broadcast_to(dv, (NBLK, 8))
        g0_ref[...] = dv * x_ref[...]

    return pl.pallas_call(
        body,
        grid=(GRID,),
        in_specs=[
            pl.BlockSpec((2, NBLK, 16), lambda n: (0, n, 0)),
            pl.BlockSpec((NBLK, 16), lambda n: (n, 0)),
        ],
        out_specs=[
            pl.BlockSpec((NBLK, 8), lambda n: (n, 0)),
            pl.BlockSpec((NBLK, 16), lambda n: (n, 0)),
        ],
        out_shape=[
            jax.ShapeDtypeStruct((NP, 8), jnp.float32),
            jax.ShapeDtypeStruct((NP, 16), jnp.float32),
        ],
    )(degp, x_pad)


def _tc_layer1(dinv, aggp, g0, w, b):
    def body(dinv_ref, aggp_ref, g0_ref, w_ref, b_ref, out_ref):
        dv = dinv_ref[:, 0:1]
        p = dv * (aggp_ref[0] + aggp_ref[1] + g0_ref[...])
        h = jnp.dot(p, w_ref[...], preferred_element_type=jnp.float32)
        h = jnp.maximum(h + b_ref[...], 0.0)
        g = dv * h
        for c in range(4):
            out_ref[c] = g[:, c * 16:(c + 1) * 16]

    return pl.pallas_call(
        body,
        grid=(GRID,),
        in_specs=[
            pl.BlockSpec((NBLK, 8), lambda n: (n, 0)),
            pl.BlockSpec((2, NBLK, 16), lambda n: (0, n, 0)),
            pl.BlockSpec((NBLK, 16), lambda n: (n, 0)),
            pl.BlockSpec((16, 64), lambda n: (0, 0)),
            pl.BlockSpec((1, 64), lambda n: (0, 0)),
        ],
        out_specs=pl.BlockSpec((4, NBLK, 16), lambda n: (0, n, 0)),
        out_shape=jax.ShapeDtypeStruct((4, NP, 16), jnp.float32),
    )(dinv, aggp, g0, w, b)


def _tc_layer2(dinv, agg, g1, w, b):
    def body(dinv_ref, agg_ref, g1_ref, w_ref, b_ref, out_ref):
        dv = dinv_ref[:, 0:1]
        p = jnp.concatenate(
            [dv * (agg_ref[c] + g1_ref[c]) for c in range(4)], axis=1)
        h = jnp.dot(p, w_ref[...], preferred_element_type=jnp.float32)
        h = jnp.maximum(h + b_ref[...], 0.0)
        g = dv * h
        for c in range(8):
            out_ref[c] = g[:, c * 16:(c + 1) * 16]

    return pl.pallas_call(
        body,
        grid=(GRID,),
        in_specs=[
            pl.BlockSpec((NBLK, 8), lambda n: (n, 0)),
            pl.BlockSpec((4, NBLK, 16), lambda n: (0, n, 0)),
            pl.BlockSpec((4, NBLK, 16), lambda n: (0, n, 0)),
            pl.BlockSpec((64, 128), lambda n: (0, 0)),
            pl.BlockSpec((1, 128), lambda n: (0, 0)),
        ],
        out_specs=pl.BlockSpec((8, NBLK, 16), lambda n: (0, n, 0)),
        out_shape=jax.ShapeDtypeStruct((8, NP, 16), jnp.float32),
    )(dinv, agg, g1, w, b)


def _tc_layer3_head(dinv, agg, g2, w3, b3, fw1, fb1, fw2, fb2):
    def body(dinv_ref, agg_ref, g2_ref, w3_ref, b3_ref, fw1_ref, fb1_ref,
             fw2_ref, fb2_ref, out_ref):
        dv = dinv_ref[:, 0:1]
        p = jnp.concatenate(
            [dv * (agg_ref[c] + g2_ref[c]) for c in range(8)], axis=1)
        h = jnp.dot(p, w3_ref[...], preferred_element_type=jnp.float32)
        h = jnp.maximum(h + b3_ref[...], 0.0)
        z = jnp.dot(h, fw1_ref[...], preferred_element_type=jnp.float32)
        z = jnp.maximum(z + fb1_ref[...], 0.0)
        o = jnp.dot(z, fw2_ref[...], preferred_element_type=jnp.float32)
        out_ref[...] = o + fb2_ref[...]

    return pl.pallas_call(
        body,
        grid=(GRID,),
        in_specs=[
            pl.BlockSpec((NBLK, 8), lambda n: (n, 0)),
            pl.BlockSpec((8, NBLK, 16), lambda n: (0, n, 0)),
            pl.BlockSpec((8, NBLK, 16), lambda n: (0, n, 0)),
            pl.BlockSpec((128, 128), lambda n: (0, 0)),
            pl.BlockSpec((1, 128), lambda n: (0, 0)),
            pl.BlockSpec((128, 64), lambda n: (0, 0)),
            pl.BlockSpec((1, 64), lambda n: (0, 0)),
            pl.BlockSpec((64, 8), lambda n: (0, 0)),
            pl.BlockSpec((1, 8), lambda n: (0, 0)),
        ],
        out_specs=pl.BlockSpec((NBLK, 8), lambda n: (n, 0)),
        out_shape=jax.ShapeDtypeStruct((NP, 8), jnp.float32),
    )(dinv, agg, g2, w3, b3, fw1, fb1, fw2, fb2)


def kernel(x, edge_index, W1, b1, W2, b2, W3, b3, fW1, fb1, fW2, fb2):
    fill = jnp.full((EP - E,), NP - 1, jnp.int32)
    src2 = jnp.concatenate([edge_index[0], fill]).reshape(EP // B, B)
    dst2 = jnp.concatenate([edge_index[1], fill]).reshape(EP // B, B)
    x_pad = jnp.zeros((NP, 16), jnp.float32).at[:N, :3].set(x)
    w1p = jnp.zeros((16, 64), jnp.float32).at[:3].set(W1)
    fw2p = jnp.zeros((64, 8), jnp.float32).at[:, :2].set(fW2)
    fb2p = jnp.zeros((8,), jnp.float32).at[:2].set(fb2)

    degp = _sc_deg(dst2)
    dinv, g0 = _tc_prep(degp.reshape(2, NP, 16), x_pad)

    agg0 = _sc_agg1(src2, dst2, g0)
    g1 = _tc_layer1(dinv, agg0.reshape(2, NP, 16), g0,
                    w1p, b1.reshape(1, 64))

    agg1 = _sc_agg4(src2, dst2, g1.reshape(4 * NP, 16))
    g2 = _tc_layer2(dinv, agg1.reshape(4, NP, 16), g1,
                    W2, b2.reshape(1, 128))

    agg2 = _sc_agg8(src2, dst2, g2.reshape(8 * NP, 16))
    outp = _tc_layer3_head(dinv, agg2.reshape(8, NP, 16), g2,
                           W3, b3.reshape(1, 128),
                           fW1, fb1.reshape(1, 64),
                           fw2p, fb2p.reshape(1, 8))
    return outp[:N, :2]


# speedup vs baseline: 13.4750x; 1.3723x over previous
import functools

import jax
import jax.numpy as jnp
from jax import lax
from jax.experimental import pallas as pl
from jax.experimental.pallas import tpu as pltpu
from jax.experimental.pallas import tpu_sc as plsc

N = 50000
NP = 50176
E = 800000
EP = 802816
NBLK = 512
GRID = NP // NBLK
RPT = NP // 16
B = 128
KI = 4
ZR = RPT // 4

_mesh = plsc.VectorSubcoreMesh(core_axis_name="c", subcore_axis_name="s")


def _zero_fill(ref, rows, width):
    zero16 = jnp.zeros((16,), jnp.float32)
    per_row = width // 16

    def body(i, _):
        r = i // per_row
        c = i % per_row
        ref[r, pl.ds(c * 16, 16)] = zero16
        return 0

    lax.fori_loop(0, rows * per_row, body, 0)


def _make_sc_agg(C, D, gather):
    partial = C == 1
    cpc = 1 if partial else C // 2
    ne_tile = EP // 32 if partial else EP // 16
    outer = ne_tile // (KI * B)

    def body(*refs):
        if gather:
            (src2, dst2, g_h, out_h, agg, src_i, dst_i, gbuf, zbuf,
             sem, ssem) = refs
        else:
            dst2, out_h, agg, src_i, dst_i, gbuf, zbuf, sem, ssem = refs
        cid = lax.axis_index("c")
        sid = lax.axis_index("s")
        row0 = sid * RPT

        _zero_fill(zbuf, ZR, D)
        if not gather:
            one16 = jnp.ones((16,), jnp.float32)

            def ones_body(i, _):
                r = i // (D // 16)
                c = i % (D // 16)
                gbuf[0, r, pl.ds(c * 16, 16)] = one16
                return 0

            lax.fori_loop(0, KI * B * (D // 16), ones_body, 0)

        for r in range(cpc):
            chunk = cid * cpc + r
            base_row = chunk * NP
            if partial:
                er0 = (cid * 16 + sid) * (ne_tile // B)
            else:
                er0 = sid * (ne_tile // B)

            for z in range(4):
                pltpu.sync_copy(zbuf, agg.at[pl.ds(row0 + z * ZR, ZR), :])
            plsc.subcore_barrier()

            if gather:

                def fire_gathers(o, p):
                    eb = er0 + o * KI
                    pltpu.sync_copy(src2.at[pl.ds(eb, KI), :],
                                    src_i.at[p])
                    if not partial:
                        def adj(k, _):
                            j = k // (B // 16)
                            kk = k % (B // 16)
                            v = src_i[p, j, pl.ds(kk * 16, 16)]
                            src_i[p, j, pl.ds(kk * 16, 16)] = v + base_row
                            return 0

                        lax.fori_loop(0, KI * (B // 16), adj, 0)
                    for j in range(KI):
                        pltpu.async_copy(
                            g_h.at[src_i.at[p, j]],
                            gbuf.at[p, pl.ds(j * B, B), :],
                            sem,
                        )

                def wait_gathers(p):
                    pltpu.make_async_copy(
                        g_h.at[pl.ds(0, KI * B), :], gbuf.at[p], sem).wait()

                def fire_scatters(o, p):
                    eb = er0 + o * KI
                    pltpu.sync_copy(dst2.at[pl.ds(eb, KI), :],
                                    dst_i.at[p])
                    for j in range(KI):
                        pltpu.async_copy(
                            gbuf.at[p, pl.ds(j * B, B), :],
                            agg.at[dst_i.at[p, j]],
                            ssem,
                            add=True,
                        )

                def wait_scatters(p):
                    pltpu.make_async_copy(
                        g_h.at[pl.ds(0, KI * B), :], gbuf.at[p], ssem).wait()

                fire_gathers(0, 0)

                def outer_body(o, _):
                    p = lax.rem(o, 2)

                    @pl.when(o >= 1)
                    def _():
                        wait_scatters(1 - p)

                    @pl.when(o + 1 < outer)
                    def _():
                        fire_gathers(o + 1, 1 - p)

                    wait_gathers(p)
                    fire_scatters(o, p)
                    return 0

                lax.fori_loop(0, outer, outer_body, 0)
                wait_scatters(lax.rem(outer - 1, 2))
            else:
                def outer_body(o, _):
                    eb = er0 + o * KI
                    pltpu.sync_copy(dst2.at[pl.ds(eb, KI), :], dst_i.at[0])
                    for j in range(KI):
                        pltpu.sync_copy(
                            gbuf.at[0, pl.ds(j * B, B), :],
                            agg.at[dst_i.at[0, j]],
                            add=True,
                        )
                    return 0

                lax.fori_loop(0, outer, outer_body, 0)
            plsc.subcore_barrier()

            out_base = (cid if partial else chunk) * NP
            pltpu.sync_copy(
                agg.at[pl.ds(row0, RPT), :],
                out_h.at[pl.ds(out_base + row0, RPT), :],
            )

    n_out = (2 if partial else C) * NP
    return functools.partial(
        pl.kernel,
        out_type=jax.ShapeDtypeStruct((n_out, D), jnp.float32),
        mesh=_mesh,
        scratch_types=[
            pltpu.VMEM_SHARED((NP, D), jnp.float32),
            pltpu.VMEM((2, KI, B), jnp.int32),
            pltpu.VMEM((2, KI, B), jnp.int32),
            pltpu.VMEM((2, KI * B, D), jnp.float32),
            pltpu.VMEM((ZR, D), jnp.float32),
            pltpu.SemaphoreType.DMA,
            pltpu.SemaphoreType.DMA,
        ],
        compiler_params=pltpu.CompilerParams(use_tc_tiling_on_sc=False),
    )(body)


_sc_deg = _make_sc_agg(1, 16, gather=False)
_sc_agg1 = _make_sc_agg(1, 16, gather=True)
_sc_agg4 = _make_sc_agg(4, 16, gather=True)
_sc_agg8 = _make_sc_agg(8, 16, gather=True)


def _tc_prep(degp, x_pad):
    def body(degp_ref, x_ref, dinv_ref, g0_ref):
        deg = degp_ref[0, :, 0:1] + degp_ref[1, :, 0:1] + 1.0
        dv = lax.rsqrt(deg)
        dinv_ref[...] = jnp.broadcast_to(dv, (NBLK, 8))
        g0_ref[...] = dv * x_ref[...]

    return pl.pallas_call(
        body,
        grid=(GRID,),
        in_specs=[
            pl.BlockSpec((2, NBLK, 16), lambda n: (0, n, 0)),
            pl.BlockSpec((NBLK, 16), lambda n: (n, 0)),
        ],
        out_specs=[
            pl.BlockSpec((NBLK, 8), lambda n: (n, 0)),
            pl.BlockSpec((NBLK, 16), lambda n: (n, 0)),
        ],
        out_shape=[
            jax.ShapeDtypeStruct((NP, 8), jnp.float32),
            jax.ShapeDtypeStruct((NP, 16), jnp.float32),
        ],
    )(degp, x_pad)


def _tc_layer1(dinv, aggp, g0, w, b):
    def body(dinv_ref, aggp_ref, g0_ref, w_ref, b_ref, out_ref):
        dv = dinv_ref[:, 0:1]
        p = dv * (aggp_ref[0] + aggp_ref[1] + g0_ref[...])
        h = jnp.dot(p, w_ref[...], preferred_element_type=jnp.float32)
        h = jnp.maximum(h + b_ref[...], 0.0)
        g = dv * h
        for c in range(4):
            out_ref[c] = g[:, c * 16:(c + 1) * 16]

    return pl.pallas_call(
        body,
        grid=(GRID,),
        in_specs=[
            pl.BlockSpec((NBLK, 8), lambda n: (n, 0)),
            pl.BlockSpec((2, NBLK, 16), lambda n: (0, n, 0)),
            pl.BlockSpec((NBLK, 16), lambda n: (n, 0)),
            pl.BlockSpec((16, 64), lambda n: (0, 0)),
            pl.BlockSpec((1, 64), lambda n: (0, 0)),
        ],
        out_specs=pl.BlockSpec((4, NBLK, 16), lambda n: (0, n, 0)),
        out_shape=jax.ShapeDtypeStruct((4, NP, 16), jnp.float32),
    )(dinv, aggp, g0, w, b)


def _tc_layer2(dinv, agg, g1, w, b):
    def body(dinv_ref, agg_ref, g1_ref, w_ref, b_ref, out_ref):
        dv = dinv_ref[:, 0:1]
        p = jnp.concatenate(
            [dv * (agg_ref[c] + g1_ref[c]) for c in range(4)], axis=1)
        h = jnp.dot(p, w_ref[...], preferred_element_type=jnp.float32)
        h = jnp.maximum(h + b_ref[...], 0.0)
        g = dv * h
        for c in range(8):
            out_ref[c] = g[:, c * 16:(c + 1) * 16]

    return pl.pallas_call(
        body,
        grid=(GRID,),
        in_specs=[
            pl.BlockSpec((NBLK, 8), lambda n: (n, 0)),
            pl.BlockSpec((4, NBLK, 16), lambda n: (0, n, 0)),
            pl.BlockSpec((4, NBLK, 16), lambda n: (0, n, 0)),
            pl.BlockSpec((64, 128), lambda n: (0, 0)),
            pl.BlockSpec((1, 128), lambda n: (0, 0)),
        ],
        out_specs=pl.BlockSpec((8, NBLK, 16), lambda n: (0, n, 0)),
        out_shape=jax.ShapeDtypeStruct((8, NP, 16), jnp.float32),
    )(dinv, agg, g1, w, b)


def _tc_layer3_head(dinv, agg, g2, w3, b3, fw1, fb1, fw2, fb2):
    def body(dinv_ref, agg_ref, g2_ref, w3_ref, b3_ref, fw1_ref, fb1_ref,
             fw2_ref, fb2_ref, out_ref):
        dv = dinv_ref[:, 0:1]
        p = jnp.concatenate(
            [dv * (agg_ref[c] + g2_ref[c]) for c in range(8)], axis=1)
        h = jnp.dot(p, w3_ref[...], preferred_element_type=jnp.float32)
        h = jnp.maximum(h + b3_ref[...], 0.0)
        z = jnp.dot(h, fw1_ref[...], preferred_element_type=jnp.float32)
        z = jnp.maximum(z + fb1_ref[...], 0.0)
        o = jnp.dot(z, fw2_ref[...], preferred_element_type=jnp.float32)
        out_ref[...] = o + fb2_ref[...]

    return pl.pallas_call(
        body,
        grid=(GRID,),
        in_specs=[
            pl.BlockSpec((NBLK, 8), lambda n: (n, 0)),
            pl.BlockSpec((8, NBLK, 16), lambda n: (0, n, 0)),
            pl.BlockSpec((8, NBLK, 16), lambda n: (0, n, 0)),
            pl.BlockSpec((128, 128), lambda n: (0, 0)),
            pl.BlockSpec((1, 128), lambda n: (0, 0)),
            pl.BlockSpec((128, 64), lambda n: (0, 0)),
            pl.BlockSpec((1, 64), lambda n: (0, 0)),
            pl.BlockSpec((64, 8), lambda n: (0, 0)),
            pl.BlockSpec((1, 8), lambda n: (0, 0)),
        ],
        out_specs=pl.BlockSpec((NBLK, 8), lambda n: (n, 0)),
        out_shape=jax.ShapeDtypeStruct((NP, 8), jnp.float32),
    )(dinv, agg, g2, w3, b3, fw1, fb1, fw2, fb2)


def kernel(x, edge_index, W1, b1, W2, b2, W3, b3, fW1, fb1, fW2, fb2):
    fill = jnp.full((EP - E,), NP - 1, jnp.int32)
    src2 = jnp.concatenate([edge_index[0], fill]).reshape(EP // B, B)
    dst2 = jnp.concatenate([edge_index[1], fill]).reshape(EP // B, B)
    x_pad = jnp.zeros((NP, 16), jnp.float32).at[:N, :3].set(x)
    w1p = jnp.zeros((16, 64), jnp.float32).at[:3].set(W1)
    fw2p = jnp.zeros((64, 8), jnp.float32).at[:, :2].set(fW2)
    fb2p = jnp.zeros((8,), jnp.float32).at[:2].set(fb2)

    degp = _sc_deg(dst2)
    dinv, g0 = _tc_prep(degp.reshape(2, NP, 16), x_pad)

    agg0 = _sc_agg1(src2, dst2, g0)
    g1 = _tc_layer1(dinv, agg0.reshape(2, NP, 16), g0,
                    w1p, b1.reshape(1, 64))

    agg1 = _sc_agg4(src2, dst2, g1.reshape(4 * NP, 16))
    g2 = _tc_layer2(dinv, agg1.reshape(4, NP, 16), g1,
                    W2, b2.reshape(1, 128))

    agg2 = _sc_agg8(src2, dst2, g2.reshape(8 * NP, 16))
    outp = _tc_layer3_head(dinv, agg2.reshape(8, NP, 16), g2,
                           W3, b3.reshape(1, 128),
                           fW1, fb1.reshape(1, 64),
                           fw2p, fb2p.reshape(1, 8))
    return outp[:N, :2]


# speedup vs baseline: 17.3933x; 1.2908x over previous
import functools

import jax
import jax.numpy as jnp
from jax import lax
from jax.experimental import pallas as pl
from jax.experimental.pallas import tpu as pltpu
from jax.experimental.pallas import tpu_sc as plsc

N = 50000
NP = 50176
E = 800000
EP = 802816
NBLK = 512
GRID = NP // NBLK
RPT = NP // 16
B = 128
KI = 4
ZR = RPT // 4

_mesh = plsc.VectorSubcoreMesh(core_axis_name="c", subcore_axis_name="s")


def _zero_fill(ref, rows, width):
    zero16 = jnp.zeros((16,), jnp.float32)
    per_row = width // 16

    def body(i, _):
        r = i // per_row
        c = i % per_row
        ref[r, pl.ds(c * 16, 16)] = zero16
        return 0

    lax.fori_loop(0, rows * per_row, body, 0)


def _make_sc_agg(C, D, gather):
    partial = C == 1
    cpc = 1 if partial else C // 2
    ne_tile = EP // 32 if partial else EP // 16
    outer = ne_tile // (KI * B)

    def body(*refs):
        if gather:
            (src2, dst2, g_h, out_h, agg, src_i, dst_i, gbuf, zbuf,
             sem, ssem) = refs
        else:
            dst2, out_h, agg, src_i, dst_i, gbuf, zbuf, sem, ssem = refs
        cid = lax.axis_index("c")
        sid = lax.axis_index("s")
        row0 = sid * RPT

        _zero_fill(zbuf, ZR, D)
        if not gather:
            one16 = jnp.ones((16,), jnp.float32)

            def ones_body(i, _):
                r = i // (D // 16)
                c = i % (D // 16)
                gbuf[0, r, pl.ds(c * 16, 16)] = one16
                return 0

            lax.fori_loop(0, KI * B * (D // 16), ones_body, 0)

        for r in range(cpc):
            chunk = cid * cpc + r
            if partial:
                er0 = (cid * 16 + sid) * (ne_tile // B)
            else:
                er0 = sid * (ne_tile // B)

            for z in range(4):
                pltpu.sync_copy(zbuf, agg.at[pl.ds(row0 + z * ZR, ZR), :])
            plsc.subcore_barrier()

            if gather:

                def fire_gathers(o, p):
                    eb = er0 + o * KI
                    pltpu.sync_copy(src2.at[pl.ds(eb, KI), :],
                                    src_i.at[p])

                    gchunk = 0 if partial else chunk

                    def adj(k, _):
                        j = k // (B // 16)
                        kk = k % (B // 16)
                        v = src_i[p, j, pl.ds(kk * 16, 16)]
                        src_i[p, j, pl.ds(kk * 16, 16)] = v * 8 + gchunk
                        return 0

                    lax.fori_loop(0, KI * (B // 16), adj, 0)
                    for j in range(KI):
                        pltpu.async_copy(
                            g_h.at[src_i.at[p, j]],
                            gbuf.at[p, pl.ds(j * B, B), :],
                            sem,
                        )

                def wait_gathers(p):
                    pltpu.make_async_copy(
                        g_h.at[pl.ds(0, KI * B), :], gbuf.at[p], sem).wait()

                def fire_scatters(o, p):
                    eb = er0 + o * KI
                    pltpu.sync_copy(dst2.at[pl.ds(eb, KI), :],
                                    dst_i.at[p])
                    for j in range(KI):
                        pltpu.async_copy(
                            gbuf.at[p, pl.ds(j * B, B), :],
                            agg.at[dst_i.at[p, j]],
                            ssem,
                            add=True,
                        )

                def wait_scatters(p):
                    pltpu.make_async_copy(
                        g_h.at[pl.ds(0, KI * B), :], gbuf.at[p], ssem).wait()

                fire_gathers(0, 0)

                def outer_body(o, _):
                    p = lax.rem(o, 2)

                    @pl.when(o >= 1)
                    def _():
                        wait_scatters(1 - p)

                    @pl.when(o + 1 < outer)
                    def _():
                        fire_gathers(o + 1, 1 - p)

                    wait_gathers(p)
                    fire_scatters(o, p)
                    return 0

                lax.fori_loop(0, outer, outer_body, 0)
                wait_scatters(lax.rem(outer - 1, 2))
            else:
                def outer_body(o, _):
                    eb = er0 + o * KI
                    pltpu.sync_copy(dst2.at[pl.ds(eb, KI), :], dst_i.at[0])
                    for j in range(KI):
                        pltpu.sync_copy(
                            gbuf.at[0, pl.ds(j * B, B), :],
                            agg.at[dst_i.at[0, j]],
                            add=True,
                        )
                    return 0

                lax.fori_loop(0, outer, outer_body, 0)
            plsc.subcore_barrier()

            out_col = ((cid if partial else chunk)) * 16
            pltpu.sync_copy(
                agg.at[pl.ds(row0, RPT), :],
                out_h.at[pl.ds(row0, RPT), pl.ds(out_col, 16)],
            )

    return functools.partial(
        pl.kernel,
        out_type=jax.ShapeDtypeStruct((NP, 128), jnp.float32),
        mesh=_mesh,
        scratch_types=[
            pltpu.VMEM_SHARED((NP, D), jnp.float32),
            pltpu.VMEM((2, KI, B), jnp.int32),
            pltpu.VMEM((2, KI, B), jnp.int32),
            pltpu.VMEM((2, KI * B, D), jnp.float32),
            pltpu.VMEM((ZR, D), jnp.float32),
            pltpu.SemaphoreType.DMA,
            pltpu.SemaphoreType.DMA,
        ],
        compiler_params=pltpu.CompilerParams(use_tc_tiling_on_sc=False),
    )(body)


_sc_deg = _make_sc_agg(1, 16, gather=False)
_sc_agg1 = _make_sc_agg(1, 16, gather=True)
_sc_agg4 = _make_sc_agg(4, 16, gather=True)
_sc_agg8 = _make_sc_agg(8, 16, gather=True)


def _tc_prep(degf, x_pad):
    def body(degf_ref, x_ref, out_ref):
        deg = degf_ref[:, 0:1] + degf_ref[:, 16:17] + 1.0
        dv = lax.rsqrt(deg)
        g0 = dv * x_ref[...]
        out_ref[...] = jnp.concatenate(
            [g0, jnp.broadcast_to(dv, (NBLK, 112))], axis=1)

    return pl.pallas_call(
        body,
        grid=(GRID,),
        in_specs=[
            pl.BlockSpec((NBLK, 128), lambda n: (n, 0)),
            pl.BlockSpec((NBLK, 16), lambda n: (n, 0)),
        ],
        out_specs=pl.BlockSpec((NBLK, 128), lambda n: (n, 0)),
        out_shape=jax.ShapeDtypeStruct((NP, 128), jnp.float32),
    )(degf, x_pad)


def _tc_layer1(aggf, f0, w, b):
    def body(agg_ref, f0_ref, w_ref, b_ref, out_ref):
        dv = f0_ref[:, 16:17]
        p = dv * (agg_ref[:, 0:16] + agg_ref[:, 16:32] + f0_ref[:, 0:16])
        h = jnp.dot(p, w_ref[...], preferred_element_type=jnp.float32)
        h = jnp.maximum(h + b_ref[...], 0.0)
        g = dv * h
        out_ref[...] = jnp.concatenate(
            [g, jnp.broadcast_to(dv, (NBLK, 64))], axis=1)

    return pl.pallas_call(
        body,
        grid=(GRID,),
        in_specs=[
            pl.BlockSpec((NBLK, 128), lambda n: (n, 0)),
            pl.BlockSpec((NBLK, 128), lambda n: (n, 0)),
            pl.BlockSpec((16, 64), lambda n: (0, 0)),
            pl.BlockSpec((1, 64), lambda n: (0, 0)),
        ],
        out_specs=pl.BlockSpec((NBLK, 128), lambda n: (n, 0)),
        out_shape=jax.ShapeDtypeStruct((NP, 128), jnp.float32),
    )(aggf, f0, w, b)


def _tc_layer2(aggf, f1, w, b):
    def body(agg_ref, f1_ref, w_ref, b_ref, out_ref):
        dv = f1_ref[:, 64:65]
        p = dv * (agg_ref[:, 0:64] + f1_ref[:, 0:64])
        h = jnp.dot(p, w_ref[...], preferred_element_type=jnp.float32)
        h = jnp.maximum(h + b_ref[...], 0.0)
        out_ref[...] = dv * h

    return pl.pallas_call(
        body,
        grid=(GRID,),
        in_specs=[
            pl.BlockSpec((NBLK, 128), lambda n: (n, 0)),
            pl.BlockSpec((NBLK, 128), lambda n: (n, 0)),
            pl.BlockSpec((64, 128), lambda n: (0, 0)),
            pl.BlockSpec((1, 128), lambda n: (0, 0)),
        ],
        out_specs=pl.BlockSpec((NBLK, 128), lambda n: (n, 0)),
        out_shape=jax.ShapeDtypeStruct((NP, 128), jnp.float32),
    )(aggf, f1, w, b)


def _tc_layer3_head(f1, aggf, f2, w3, b3, fw1, fb1, fw2, fb2):
    def body(f1_ref, agg_ref, f2_ref, w3_ref, b3_ref, fw1_ref, fb1_ref,
             fw2_ref, fb2_ref, out_ref):
        dv = f1_ref[:, 64:65]
        p = dv * (agg_ref[...] + f2_ref[...])
        h = jnp.dot(p, w3_ref[...], preferred_element_type=jnp.float32)
        h = jnp.maximum(h + b3_ref[...], 0.0)
        z = jnp.dot(h, fw1_ref[...], preferred_element_type=jnp.float32)
        z = jnp.maximum(z + fb1_ref[...], 0.0)
        o = jnp.dot(z, fw2_ref[...], preferred_element_type=jnp.float32)
        out_ref[...] = o + fb2_ref[...]

    return pl.pallas_call(
        body,
        grid=(GRID,),
        in_specs=[
            pl.BlockSpec((NBLK, 128), lambda n: (n, 0)),
            pl.BlockSpec((NBLK, 128), lambda n: (n, 0)),
            pl.BlockSpec((NBLK, 128), lambda n: (n, 0)),
            pl.BlockSpec((128, 128), lambda n: (0, 0)),
            pl.BlockSpec((1, 128), lambda n: (0, 0)),
            pl.BlockSpec((128, 64), lambda n: (0, 0)),
            pl.BlockSpec((1, 64), lambda n: (0, 0)),
            pl.BlockSpec((64, 8), lambda n: (0, 0)),
            pl.BlockSpec((1, 8), lambda n: (0, 0)),
        ],
        out_specs=pl.BlockSpec((NBLK, 8), lambda n: (n, 0)),
        out_shape=jax.ShapeDtypeStruct((NP, 8), jnp.float32),
    )(f1, aggf, f2, w3, b3, fw1, fb1, fw2, fb2)


def kernel(x, edge_index, W1, b1, W2, b2, W3, b3, fW1, fb1, fW2, fb2):
    fill = jnp.full((EP - E,), NP - 1, jnp.int32)
    src2 = jnp.concatenate([edge_index[0], fill]).reshape(EP // B, B)
    dst2 = jnp.concatenate([edge_index[1], fill]).reshape(EP // B, B)
    x_pad = jnp.zeros((NP, 16), jnp.float32).at[:N, :3].set(x)
    w1p = jnp.zeros((16, 64), jnp.float32).at[:3].set(W1)
    fw2p = jnp.zeros((64, 8), jnp.float32).at[:, :2].set(fW2)
    fb2p = jnp.zeros((8,), jnp.float32).at[:2].set(fb2)

    degf = _sc_deg(dst2)
    f0 = _tc_prep(degf, x_pad)

    agg0 = _sc_agg1(src2, dst2, f0.reshape(NP * 8, 16))
    f1 = _tc_layer1(agg0, f0, w1p, b1.reshape(1, 64))

    agg1 = _sc_agg4(src2, dst2, f1.reshape(NP * 8, 16))
    f2 = _tc_layer2(agg1, f1, W2, b2.reshape(1, 128))

    agg2 = _sc_agg8(src2, dst2, f2.reshape(NP * 8, 16))
    outp = _tc_layer3_head(f1, agg2, f2,
                           W3, b3.reshape(1, 128),
                           fW1, fb1.reshape(1, 64),
                           fw2p, fb2p.reshape(1, 8))
    return outp[:N, :2]


# speedup vs baseline: 23.0804x; 1.3270x over previous
import functools

import jax
import jax.numpy as jnp
from jax import lax
from jax.experimental import pallas as pl
from jax.experimental.pallas import tpu as pltpu
from jax.experimental.pallas import tpu_sc as plsc

N = 50000
NP = 50176
E = 800000
EP = 802816
NBLK = 1024
GRID = NP // NBLK
RPT = NP // 16
B = 128
KI = 2

_mesh = plsc.VectorSubcoreMesh(core_axis_name="c", subcore_axis_name="s")


def _make_sc_agg(C, D, gather):
    partial = C == 1
    cpc = 1 if partial else C // 2
    ne_tile = EP // 32 if partial else EP // 16
    outer = ne_tile // (KI * B)
    S = 128 // D

    def body(*refs):
        if gather:
            (epk2, g_h, out_h, agg, src_i, dst_i, gbuf,
             sem, ssem) = refs
        else:
            epk2, out_h, agg, src_i, dst_i, gbuf, sem, ssem = refs
        cid = lax.axis_index("c")
        sid = lax.axis_index("s")
        row0 = sid * RPT
        nz_full, nz_rem = divmod(RPT, KI * B)

        def fill_gbuf0(val16):
            def fbody(i, _):
                rr = i // (D // 16)
                cc = i % (D // 16)
                gbuf[0, rr, pl.ds(cc * 16, 16)] = val16
                return 0

            lax.fori_loop(0, KI * B * (D // 16), fbody, 0)

        for r in range(cpc):
            chunk = cid * cpc + r
            if partial:
                er0 = (cid * 16 + sid) * (ne_tile // B)
            else:
                er0 = sid * (ne_tile // B)

            fill_gbuf0(jnp.zeros((16,), jnp.float32))
            for z in range(nz_full):
                pltpu.sync_copy(
                    gbuf.at[0],
                    agg.at[pl.ds(row0 + z * KI * B, KI * B), :])
            if nz_rem:
                pltpu.sync_copy(
                    gbuf.at[0, pl.ds(0, nz_rem), :],
                    agg.at[pl.ds(row0 + nz_full * KI * B, nz_rem), :])
            if not gather:
                fill_gbuf0(jnp.ones((16,), jnp.float32))
            plsc.subcore_barrier()

            if gather:

                def fire_gathers(o, p):
                    eb = er0 + o * KI
                    pltpu.sync_copy(epk2.at[pl.ds(eb, KI), :],
                                    src_i.at[p])

                    gchunk = 0 if partial else chunk

                    def adj(k, _):
                        j = k // (B // 16)
                        kk = k % (B // 16)
                        pk = src_i[p, j, pl.ds(kk * 16, 16)]
                        dst_i[p, j, pl.ds(kk * 16, 16)] = (pk >> 16) & 0xFFFF
                        src_i[p, j, pl.ds(kk * 16, 16)] = (
                            (pk & 0xFFFF) * S + gchunk)
                        return 0

                    lax.fori_loop(0, KI * (B // 16), adj, 0)
                    for j in range(KI):
                        pltpu.async_copy(
                            g_h.at[src_i.at[p, j]],
                            gbuf.at[p, pl.ds(j * B, B), :],
                            sem,
                        )

                def wait_gathers(p):
                    pltpu.make_async_copy(
                        g_h.at[pl.ds(0, KI * B), :], gbuf.at[p], sem).wait()

                def fire_scatters(o, p):
                    for j in range(KI):
                        pltpu.async_copy(
                            gbuf.at[p, pl.ds(j * B, B), :],
                            agg.at[dst_i.at[p, j]],
                            ssem,
                            add=True,
                        )

                def wait_scatters(p):
                    pltpu.make_async_copy(
                        g_h.at[pl.ds(0, KI * B), :], gbuf.at[p], ssem).wait()

                fire_gathers(0, 0)

                def outer_body(o, _):
                    p = lax.rem(o, 2)

                    @pl.when(o >= 1)
                    def _():
                        wait_scatters(1 - p)

                    @pl.when(o + 1 < outer)
                    def _():
                        fire_gathers(o + 1, 1 - p)

                    wait_gathers(p)
                    fire_scatters(o, p)
                    return 0

                lax.fori_loop(0, outer, outer_body, 0)
                wait_scatters(lax.rem(outer - 1, 2))
            else:
                def outer_body(o, _):
                    eb = er0 + o * KI
                    pltpu.sync_copy(epk2.at[pl.ds(eb, KI), :], dst_i.at[0])

                    def adj(k, _):
                        j = k // (B // 16)
                        kk = k % (B // 16)
                        pk = dst_i[0, j, pl.ds(kk * 16, 16)]
                        dst_i[0, j, pl.ds(kk * 16, 16)] = (pk >> 16) & 0xFFFF
                        return 0

                    lax.fori_loop(0, KI * (B // 16), adj, 0)
                    for j in range(KI):
                        pltpu.sync_copy(
                            gbuf.at[0, pl.ds(j * B, B), :],
                            agg.at[dst_i.at[0, j]],
                            add=True,
                        )
                    return 0

                lax.fori_loop(0, outer, outer_body, 0)
            plsc.subcore_barrier()

            out_col = ((cid if partial else chunk)) * D
            pltpu.sync_copy(
                agg.at[pl.ds(row0, RPT), :],
                out_h.at[pl.ds(row0, RPT), pl.ds(out_col, D)],
            )

    return functools.partial(
        pl.kernel,
        out_type=jax.ShapeDtypeStruct((NP, 128), jnp.float32),
        mesh=_mesh,
        scratch_types=[
            pltpu.VMEM_SHARED((NP, D), jnp.float32),
            pltpu.VMEM((2, KI, B), jnp.int32),
            pltpu.VMEM((2, KI, B), jnp.int32),
            pltpu.VMEM((2, KI * B, D), jnp.float32),
            pltpu.SemaphoreType.DMA,
            pltpu.SemaphoreType.DMA,
        ],
        compiler_params=pltpu.CompilerParams(use_tc_tiling_on_sc=False),
    )(body)


_sc_deg = _make_sc_agg(1, 16, gather=False)
_sc_agg1 = _make_sc_agg(1, 16, gather=True)
_sc_agg2 = _make_sc_agg(2, 32, gather=True)
_sc_agg4 = _make_sc_agg(4, 32, gather=True)


def _tc_prep(degf, x_pad):
    def body(degf_ref, x_ref, out_ref):
        deg = degf_ref[:, 0:1] + degf_ref[:, 16:17] + 1.0
        dv = lax.rsqrt(deg)
        g0 = dv * x_ref[...]
        out_ref[...] = jnp.concatenate(
            [g0, jnp.broadcast_to(dv, (NBLK, 112))], axis=1)

    return pl.pallas_call(
        body,
        grid=(GRID,),
        in_specs=[
            pl.BlockSpec((NBLK, 128), lambda n: (n, 0)),
            pl.BlockSpec((NBLK, 16), lambda n: (n, 0)),
        ],
        out_specs=pl.BlockSpec((NBLK, 128), lambda n: (n, 0)),
        out_shape=jax.ShapeDtypeStruct((NP, 128), jnp.float32),
    )(degf, x_pad)


def _tc_layer1(aggf, f0, w, b):
    def body(agg_ref, f0_ref, w_ref, b_ref, out_ref):
        dv = f0_ref[:, 16:17]
        p = dv * (agg_ref[:, 0:16] + agg_ref[:, 16:32] + f0_ref[:, 0:16])
        h = jnp.dot(p, w_ref[...], preferred_element_type=jnp.float32)
        h = jnp.maximum(h + b_ref[...], 0.0)
        g = dv * h
        out_ref[...] = jnp.concatenate(
            [g, jnp.broadcast_to(dv, (NBLK, 64))], axis=1)

    return pl.pallas_call(
        body,
        grid=(GRID,),
        in_specs=[
            pl.BlockSpec((NBLK, 128), lambda n: (n, 0)),
            pl.BlockSpec((NBLK, 128), lambda n: (n, 0)),
            pl.BlockSpec((16, 64), lambda n: (0, 0)),
            pl.BlockSpec((1, 64), lambda n: (0, 0)),
        ],
        out_specs=pl.BlockSpec((NBLK, 128), lambda n: (n, 0)),
        out_shape=jax.ShapeDtypeStruct((NP, 128), jnp.float32),
    )(aggf, f0, w, b)


def _tc_layer2(aggf, f1, w, b):
    def body(agg_ref, f1_ref, w_ref, b_ref, out_ref):
        dv = f1_ref[:, 64:65]
        p = dv * (agg_ref[:, 0:64] + f1_ref[:, 0:64])
        h = jnp.dot(p, w_ref[...], preferred_element_type=jnp.float32)
        h = jnp.maximum(h + b_ref[...], 0.0)
        out_ref[...] = dv * h

    return pl.pallas_call(
        body,
        grid=(GRID,),
        in_specs=[
            pl.BlockSpec((NBLK, 128), lambda n: (n, 0)),
            pl.BlockSpec((NBLK, 128), lambda n: (n, 0)),
            pl.BlockSpec((64, 128), lambda n: (0, 0)),
            pl.BlockSpec((1, 128), lambda n: (0, 0)),
        ],
        out_specs=pl.BlockSpec((NBLK, 128), lambda n: (n, 0)),
        out_shape=jax.ShapeDtypeStruct((NP, 128), jnp.float32),
    )(aggf, f1, w, b)


def _tc_layer3_head(f1, aggf, f2, w3, b3, fw1, fb1, fw2, fb2):
    def body(f1_ref, agg_ref, f2_ref, w3_ref, b3_ref, fw1_ref, fb1_ref,
             fw2_ref, fb2_ref, out_ref):
        dv = f1_ref[:, 64:65]
        p = dv * (agg_ref[...] + f2_ref[...])
        h = jnp.dot(p, w3_ref[...], preferred_element_type=jnp.float32)
        h = jnp.maximum(h + b3_ref[...], 0.0)
        z = jnp.dot(h, fw1_ref[...], preferred_element_type=jnp.float32)
        z = jnp.maximum(z + fb1_ref[...], 0.0)
        o = jnp.dot(z, fw2_ref[...], preferred_element_type=jnp.float32)
        out_ref[...] = o + fb2_ref[...]

    return pl.pallas_call(
        body,
        grid=(GRID,),
        in_specs=[
            pl.BlockSpec((NBLK, 128), lambda n: (n, 0)),
            pl.BlockSpec((NBLK, 128), lambda n: (n, 0)),
            pl.BlockSpec((NBLK, 128), lambda n: (n, 0)),
            pl.BlockSpec((128, 128), lambda n: (0, 0)),
            pl.BlockSpec((1, 128), lambda n: (0, 0)),
            pl.BlockSpec((128, 64), lambda n: (0, 0)),
            pl.BlockSpec((1, 64), lambda n: (0, 0)),
            pl.BlockSpec((64, 8), lambda n: (0, 0)),
            pl.BlockSpec((1, 8), lambda n: (0, 0)),
        ],
        out_specs=pl.BlockSpec((NBLK, 8), lambda n: (n, 0)),
        out_shape=jax.ShapeDtypeStruct((NP, 8), jnp.float32),
    )(f1, aggf, f2, w3, b3, fw1, fb1, fw2, fb2)


def kernel(x, edge_index, W1, b1, W2, b2, W3, b3, fW1, fb1, fW2, fb2):
    epk = (edge_index[1] << 16) | edge_index[0]
    fill = jnp.full((EP - E,), ((NP - 1) << 16) | (NP - 1), jnp.int32)
    epk2 = jnp.concatenate([epk, fill]).reshape(EP // B, B)
    x_pad = jnp.zeros((NP, 16), jnp.float32).at[:N, :3].set(x)
    w1p = jnp.zeros((16, 64), jnp.float32).at[:3].set(W1)
    fw2p = jnp.zeros((64, 8), jnp.float32).at[:, :2].set(fW2)
    fb2p = jnp.zeros((8,), jnp.float32).at[:2].set(fb2)

    degf = _sc_deg(epk2)
    f0 = _tc_prep(degf, x_pad)

    agg0 = _sc_agg1(epk2, f0.reshape(NP * 8, 16))
    f1 = _tc_layer1(agg0, f0, w1p, b1.reshape(1, 64))

    agg1 = _sc_agg2(epk2, f1.reshape(NP * 4, 32))
    f2 = _tc_layer2(agg1, f1, W2, b2.reshape(1, 128))

    agg2 = _sc_agg4(epk2, f2.reshape(NP * 4, 32))
    outp = _tc_layer3_head(f1, agg2, f2,
                           W3, b3.reshape(1, 128),
                           fW1, fb1.reshape(1, 64),
                           fw2p, fb2p.reshape(1, 8))
    return outp[:N, :2]


# speedup vs baseline: 24.4757x; 1.0605x over previous
import functools

import jax
import jax.numpy as jnp
from jax import lax
from jax.experimental import pallas as pl
from jax.experimental.pallas import tpu as pltpu
from jax.experimental.pallas import tpu_sc as plsc

N = 50000
NP = 50176
E = 800000
EP = 802816
NBLK = 1024
GRID = NP // NBLK
RPT = NP // 16
B = 128
KI = 2

_mesh = plsc.VectorSubcoreMesh(core_axis_name="c", subcore_axis_name="s")


def _make_sc_agg(C, D, gather, ki):
    partial = C == 1
    cpc = 1 if partial else C // 2
    ne_tile = EP // 32 if partial else EP // 16
    outer = ne_tile // (ki * B)
    S = 128 // D

    def body(*refs):
        if gather:
            (epk2, g_h, out_h, agg, src_i, dst_i, gbuf,
             sem, ssem) = refs
        else:
            epk2, out_h, agg, src_i, dst_i, gbuf, sem, ssem = refs
        cid = lax.axis_index("c")
        sid = lax.axis_index("s")
        row0 = sid * RPT
        nz_full, nz_rem = divmod(RPT, ki * B)

        def fill_gbuf0(val16):
            def fbody(i, _):
                rr = i // (D // 16)
                cc = i % (D // 16)
                gbuf[0, rr, pl.ds(cc * 16, 16)] = val16
                return 0

            lax.fori_loop(0, ki * B * (D // 16), fbody, 0)

        for r in range(cpc):
            chunk = cid * cpc + r
            if partial:
                er0 = (cid * 16 + sid) * (ne_tile // B)
            else:
                er0 = sid * (ne_tile // B)

            fill_gbuf0(jnp.zeros((16,), jnp.float32))
            for z in range(nz_full):
                pltpu.sync_copy(
                    gbuf.at[0],
                    agg.at[pl.ds(row0 + z * ki * B, ki * B), :])
            if nz_rem:
                pltpu.sync_copy(
                    gbuf.at[0, pl.ds(0, nz_rem), :],
                    agg.at[pl.ds(row0 + nz_full * ki * B, nz_rem), :])
            if not gather:
                fill_gbuf0(jnp.ones((16,), jnp.float32))
            plsc.subcore_barrier()

            if gather:

                def fire_gathers(o, p):
                    eb = er0 + o * ki
                    pltpu.sync_copy(epk2.at[pl.ds(eb, ki), :],
                                    src_i.at[p])

                    gchunk = 0 if partial else chunk

                    def adj(k, _):
                        j = k // (B // 16)
                        kk = k % (B // 16)
                        pk = src_i[p, j, pl.ds(kk * 16, 16)]
                        dst_i[p, j, pl.ds(kk * 16, 16)] = (pk >> 16) & 0xFFFF
                        src_i[p, j, pl.ds(kk * 16, 16)] = (
                            (pk & 0xFFFF) * S + gchunk)
                        return 0

                    lax.fori_loop(0, ki * (B // 16), adj, 0)
                    for j in range(ki):
                        pltpu.async_copy(
                            g_h.at[src_i.at[p, j]],
                            gbuf.at[p, pl.ds(j * B, B), :],
                            sem,
                        )

                def wait_gathers(p):
                    pltpu.make_async_copy(
                        g_h.at[pl.ds(0, ki * B), :], gbuf.at[p], sem).wait()

                def fire_scatters(o, p):
                    for j in range(ki):
                        pltpu.async_copy(
                            gbuf.at[p, pl.ds(j * B, B), :],
                            agg.at[dst_i.at[p, j]],
                            ssem,
                            add=True,
                        )

                def wait_scatters(p):
                    pltpu.make_async_copy(
                        g_h.at[pl.ds(0, ki * B), :], gbuf.at[p], ssem).wait()

                fire_gathers(0, 0)

                def outer_body(o, _):
                    p = lax.rem(o, 2)

                    @pl.when(o >= 1)
                    def _():
                        wait_scatters(1 - p)

                    @pl.when(o + 1 < outer)
                    def _():
                        fire_gathers(o + 1, 1 - p)

                    wait_gathers(p)
                    fire_scatters(o, p)
                    return 0

                lax.fori_loop(0, outer, outer_body, 0)
                wait_scatters(lax.rem(outer - 1, 2))
            else:
                def load_dst(o, p):
                    eb = er0 + o * ki
                    pltpu.sync_copy(epk2.at[pl.ds(eb, ki), :], dst_i.at[p])

                    def adj(k, _):
                        j = k // (B // 16)
                        kk = k % (B // 16)
                        pk = dst_i[p, j, pl.ds(kk * 16, 16)]
                        dst_i[p, j, pl.ds(kk * 16, 16)] = (pk >> 16) & 0xFFFF
                        return 0

                    lax.fori_loop(0, ki * (B // 16), adj, 0)

                def drain_scatters():
                    pltpu.make_async_copy(
                        out_h.at[pl.ds(0, ki * B), pl.ds(0, D)],
                        gbuf.at[0], ssem).wait()

                load_dst(0, 0)

                def outer_body(o, _):
                    p = lax.rem(o, 2)
                    for j in range(ki):
                        pltpu.async_copy(
                            gbuf.at[0, pl.ds(j * B, B), :],
                            agg.at[dst_i.at[p, j]],
                            ssem,
                            add=True,
                        )

                    @pl.when(o >= 1)
                    def _():
                        drain_scatters()

                    @pl.when(o + 1 < outer)
                    def _():
                        load_dst(o + 1, 1 - p)
                    return 0

                lax.fori_loop(0, outer, outer_body, 0)
                drain_scatters()
            plsc.subcore_barrier()

            out_col = ((cid if partial else chunk)) * D
            pltpu.sync_copy(
                agg.at[pl.ds(row0, RPT), :],
                out_h.at[pl.ds(row0, RPT), pl.ds(out_col, D)],
            )

    return functools.partial(
        pl.kernel,
        out_type=jax.ShapeDtypeStruct((NP, 128), jnp.float32),
        mesh=_mesh,
        scratch_types=[
            pltpu.VMEM_SHARED((NP, D), jnp.float32),
            pltpu.VMEM((2, ki, B), jnp.int32),
            pltpu.VMEM((2, ki, B), jnp.int32),
            pltpu.VMEM((2, ki * B, D), jnp.float32),
            pltpu.SemaphoreType.DMA,
            pltpu.SemaphoreType.DMA,
        ],
        compiler_params=pltpu.CompilerParams(use_tc_tiling_on_sc=False),
    )(body)


_sc_deg = _make_sc_agg(1, 16, gather=False, ki=4)
_sc_agg1 = _make_sc_agg(1, 16, gather=True, ki=4)
_sc_agg2 = _make_sc_agg(2, 32, gather=True, ki=2)
_sc_agg4 = _make_sc_agg(4, 32, gather=True, ki=2)


def _tc_prep(degf, x_pad):
    def body(degf_ref, x_ref, out_ref):
        deg = degf_ref[:, 0:1] + degf_ref[:, 16:17] + 1.0
        dv = lax.rsqrt(deg)
        g0 = dv * x_ref[...]
        out_ref[...] = jnp.concatenate(
            [g0, jnp.broadcast_to(dv, (NBLK, 112))], axis=1)

    return pl.pallas_call(
        body,
        grid=(GRID,),
        in_specs=[
            pl.BlockSpec((NBLK, 128), lambda n: (n, 0)),
            pl.BlockSpec((NBLK, 16), lambda n: (n, 0)),
        ],
        out_specs=pl.BlockSpec((NBLK, 128), lambda n: (n, 0)),
        out_shape=jax.ShapeDtypeStruct((NP, 128), jnp.float32),
    )(degf, x_pad)


def _tc_layer1(aggf, f0, w, b):
    def body(agg_ref, f0_ref, w_ref, b_ref, out_ref):
        dv = f0_ref[:, 16:17]
        p = dv * (agg_ref[:, 0:16] + agg_ref[:, 16:32] + f0_ref[:, 0:16])
        h = jnp.dot(p, w_ref[...], preferred_element_type=jnp.float32)
        h = jnp.maximum(h + b_ref[...], 0.0)
        g = dv * h
        out_ref[...] = jnp.concatenate(
            [g, jnp.broadcast_to(dv, (NBLK, 64))], axis=1)

    return pl.pallas_call(
        body,
        grid=(GRID,),
        in_specs=[
            pl.BlockSpec((NBLK, 128), lambda n: (n, 0)),
            pl.BlockSpec((NBLK, 128), lambda n: (n, 0)),
            pl.BlockSpec((16, 64), lambda n: (0, 0)),
            pl.BlockSpec((1, 64), lambda n: (0, 0)),
        ],
        out_specs=pl.BlockSpec((NBLK, 128), lambda n: (n, 0)),
        out_shape=jax.ShapeDtypeStruct((NP, 128), jnp.float32),
    )(aggf, f0, w, b)


def _tc_layer2(aggf, f1, w, b):
    def body(agg_ref, f1_ref, w_ref, b_ref, out_ref):
        dv = f1_ref[:, 64:65]
        p = dv * (agg_ref[:, 0:64] + f1_ref[:, 0:64])
        h = jnp.dot(p, w_ref[...], preferred_element_type=jnp.float32)
        h = jnp.maximum(h + b_ref[...], 0.0)
        out_ref[...] = dv * h

    return pl.pallas_call(
        body,
        grid=(GRID,),
        in_specs=[
            pl.BlockSpec((NBLK, 128), lambda n: (n, 0)),
            pl.BlockSpec((NBLK, 128), lambda n: (n, 0)),
            pl.BlockSpec((64, 128), lambda n: (0, 0)),
            pl.BlockSpec((1, 128), lambda n: (0, 0)),
        ],
        out_specs=pl.BlockSpec((NBLK, 128), lambda n: (n, 0)),
        out_shape=jax.ShapeDtypeStruct((NP, 128), jnp.float32),
    )(aggf, f1, w, b)


def _tc_layer3_head(f1, aggf, f2, w3, b3, fw1, fb1, fw2, fb2):
    def body(f1_ref, agg_ref, f2_ref, w3_ref, b3_ref, fw1_ref, fb1_ref,
             fw2_ref, fb2_ref, out_ref):
        dv = f1_ref[:, 64:65]
        p = dv * (agg_ref[...] + f2_ref[...])
        h = jnp.dot(p, w3_ref[...], preferred_element_type=jnp.float32)
        h = jnp.maximum(h + b3_ref[...], 0.0)
        z = jnp.dot(h, fw1_ref[...], preferred_element_type=jnp.float32)
        z = jnp.maximum(z + fb1_ref[...], 0.0)
        o = jnp.dot(z, fw2_ref[...], preferred_element_type=jnp.float32)
        out_ref[...] = o + fb2_ref[...]

    return pl.pallas_call(
        body,
        grid=(GRID,),
        in_specs=[
            pl.BlockSpec((NBLK, 128), lambda n: (n, 0)),
            pl.BlockSpec((NBLK, 128), lambda n: (n, 0)),
            pl.BlockSpec((NBLK, 128), lambda n: (n, 0)),
            pl.BlockSpec((128, 128), lambda n: (0, 0)),
            pl.BlockSpec((1, 128), lambda n: (0, 0)),
            pl.BlockSpec((128, 64), lambda n: (0, 0)),
            pl.BlockSpec((1, 64), lambda n: (0, 0)),
            pl.BlockSpec((64, 8), lambda n: (0, 0)),
            pl.BlockSpec((1, 8), lambda n: (0, 0)),
        ],
        out_specs=pl.BlockSpec((NBLK, 8), lambda n: (n, 0)),
        out_shape=jax.ShapeDtypeStruct((NP, 8), jnp.float32),
    )(f1, aggf, f2, w3, b3, fw1, fb1, fw2, fb2)


def kernel(x, edge_index, W1, b1, W2, b2, W3, b3, fW1, fb1, fW2, fb2):
    epk = (edge_index[1] << 16) | edge_index[0]
    fill = jnp.full((EP - E,), ((NP - 1) << 16) | (NP - 1), jnp.int32)
    epk2 = jnp.concatenate([epk, fill]).reshape(EP // B, B)
    x_pad = jnp.zeros((NP, 16), jnp.float32).at[:N, :3].set(x)
    w1p = jnp.zeros((16, 64), jnp.float32).at[:3].set(W1)
    fw2p = jnp.zeros((64, 8), jnp.float32).at[:, :2].set(fW2)
    fb2p = jnp.zeros((8,), jnp.float32).at[:2].set(fb2)

    degf = _sc_deg(epk2)
    f0 = _tc_prep(degf, x_pad)

    agg0 = _sc_agg1(epk2, f0.reshape(NP * 8, 16))
    f1 = _tc_layer1(agg0, f0, w1p, b1.reshape(1, 64))

    agg1 = _sc_agg2(epk2, f1.reshape(NP * 4, 32))
    f2 = _tc_layer2(agg1, f1, W2, b2.reshape(1, 128))

    agg2 = _sc_agg4(epk2, f2.reshape(NP * 4, 32))
    outp = _tc_layer3_head(f1, agg2, f2,
                           W3, b3.reshape(1, 128),
                           fW1, fb1.reshape(1, 64),
                           fw2p, fb2p.reshape(1, 8))
    return outp[:N, :2]


# speedup vs baseline: 24.9299x; 1.0186x over previous
import functools

import jax
import jax.numpy as jnp
from jax import lax
from jax.experimental import pallas as pl
from jax.experimental.pallas import tpu as pltpu
from jax.experimental.pallas import tpu_sc as plsc

N = 50000
NP = 50176
E = 800000
EP = 802816
NBLK = 1024
GRID = NP // NBLK
RPT = NP // 16
B = 128
KI = 2

_mesh = plsc.VectorSubcoreMesh(core_axis_name="c", subcore_axis_name="s")


def _make_sc_agg(C, D, gather, ki):
    partial = C == 1
    cpc = 1 if partial else C // 2
    ne_tile = EP // 32 if partial else EP // 16
    outer = ne_tile // (ki * B)
    S = 128 // D

    def body(*refs):
        if gather:
            (epk2, g_h, out_h, agg, src_i, dst_i, gbuf,
             sem, ssem) = refs
        else:
            epk2, out_h, agg, src_i, dst_i, gbuf, sem, ssem = refs
        cid = lax.axis_index("c")
        sid = lax.axis_index("s")
        row0 = sid * RPT
        nz_full, nz_rem = divmod(RPT, ki * B)

        def fill_gbuf0(val16):
            def fbody(i, _):
                rr = i // (D // 16)
                cc = i % (D // 16)
                gbuf[0, rr, pl.ds(cc * 16, 16)] = val16
                return 0

            lax.fori_loop(0, ki * B * (D // 16), fbody, 0)

        for r in range(cpc):
            chunk = cid * cpc + r
            if partial:
                er0 = (cid * 16 + sid) * (ne_tile // B)
            else:
                er0 = sid * (ne_tile // B)

            fill_gbuf0(jnp.zeros((16,), jnp.float32))
            for z in range(nz_full):
                pltpu.sync_copy(
                    gbuf.at[0],
                    agg.at[pl.ds(row0 + z * ki * B, ki * B), :])
            if nz_rem:
                pltpu.sync_copy(
                    gbuf.at[0, pl.ds(0, nz_rem), :],
                    agg.at[pl.ds(row0 + nz_full * ki * B, nz_rem), :])
            if not gather:
                fill_gbuf0(jnp.ones((16,), jnp.float32))
            plsc.subcore_barrier()

            if gather:

                def fire_gathers(o, p):
                    eb = er0 + o * ki
                    pltpu.sync_copy(epk2.at[pl.ds(eb, ki), :],
                                    src_i.at[p])

                    gchunk = 0 if partial else chunk

                    def adj(k, _):
                        j = k // (B // 16)
                        kk = k % (B // 16)
                        pk = src_i[p, j, pl.ds(kk * 16, 16)]
                        dst_i[p, j, pl.ds(kk * 16, 16)] = (pk >> 16) & 0xFFFF
                        src_i[p, j, pl.ds(kk * 16, 16)] = (
                            (pk & 0xFFFF) * S + gchunk)
                        return 0

                    lax.fori_loop(0, ki * (B // 16), adj, 0)
                    for j in range(ki):
                        pltpu.async_copy(
                            g_h.at[src_i.at[p, j]],
                            gbuf.at[p, pl.ds(j * B, B), :],
                            sem,
                        )

                def wait_gathers(p):
                    pltpu.make_async_copy(
                        g_h.at[pl.ds(0, ki * B), :], gbuf.at[p], sem).wait()

                def fire_scatters(o, p):
                    for j in range(ki):
                        pltpu.async_copy(
                            gbuf.at[p, pl.ds(j * B, B), :],
                            agg.at[dst_i.at[p, j]],
                            ssem,
                            add=True,
                        )

                def wait_scatters(p):
                    pltpu.make_async_copy(
                        g_h.at[pl.ds(0, ki * B), :], gbuf.at[p], ssem).wait()

                fire_gathers(0, 0)

                def outer_body(o, _):
                    p = lax.rem(o, 2)

                    @pl.when(o >= 1)
                    def _():
                        wait_scatters(1 - p)

                    @pl.when(o + 1 < outer)
                    def _():
                        fire_gathers(o + 1, 1 - p)

                    wait_gathers(p)
                    fire_scatters(o, p)
                    return 0

                lax.fori_loop(0, outer, outer_body, 0)
                wait_scatters(lax.rem(outer - 1, 2))
            else:
                def load_dst(o, p):
                    eb = er0 + o * ki
                    pltpu.sync_copy(epk2.at[pl.ds(eb, ki), :], dst_i.at[p])

                    def adj(k, _):
                        j = k // (B // 16)
                        kk = k % (B // 16)
                        pk = dst_i[p, j, pl.ds(kk * 16, 16)]
                        dst_i[p, j, pl.ds(kk * 16, 16)] = (pk >> 16) & 0xFFFF
                        return 0

                    lax.fori_loop(0, ki * (B // 16), adj, 0)

                def drain_scatters():
                    pltpu.make_async_copy(
                        out_h.at[pl.ds(0, ki * B), pl.ds(0, D)],
                        gbuf.at[0], ssem).wait()

                load_dst(0, 0)

                def outer_body(o, _):
                    p = lax.rem(o, 2)
                    for j in range(ki):
                        pltpu.async_copy(
                            gbuf.at[0, pl.ds(j * B, B), :],
                            agg.at[dst_i.at[p, j]],
                            ssem,
                            add=True,
                        )

                    @pl.when(o >= 1)
                    def _():
                        drain_scatters()

                    @pl.when(o + 1 < outer)
                    def _():
                        load_dst(o + 1, 1 - p)
                    return 0

                lax.fori_loop(0, outer, outer_body, 0)
                drain_scatters()
            plsc.subcore_barrier()

            out_col = ((cid if partial else chunk)) * D
            pltpu.sync_copy(
                agg.at[pl.ds(row0, RPT), :],
                out_h.at[pl.ds(row0, RPT), pl.ds(out_col, D)],
            )

    return functools.partial(
        pl.kernel,
        out_type=jax.ShapeDtypeStruct((NP, 128), jnp.float32),
        mesh=_mesh,
        scratch_types=[
            pltpu.VMEM_SHARED((NP, D), jnp.float32),
            pltpu.VMEM((2, ki, B), jnp.int32),
            pltpu.VMEM((2, ki, B), jnp.int32),
            pltpu.VMEM((2, ki * B, D), jnp.float32),
            pltpu.SemaphoreType.DMA,
            pltpu.SemaphoreType.DMA,
        ],
        compiler_params=pltpu.CompilerParams(use_tc_tiling_on_sc=False),
    )(body)


_sc_deg = _make_sc_agg(1, 16, gather=False, ki=7)
_sc_agg1 = _make_sc_agg(1, 16, gather=True, ki=7)
_sc_agg2 = _make_sc_agg(2, 32, gather=True, ki=2)
_sc_agg4 = _make_sc_agg(4, 32, gather=True, ki=2)


def _tc_prep(degf, x_pad):
    def body(degf_ref, x_ref, out_ref):
        deg = degf_ref[:, 0:1] + degf_ref[:, 16:17] + 1.0
        dv = lax.rsqrt(deg)
        g0 = dv * x_ref[...]
        out_ref[...] = jnp.concatenate(
            [g0, jnp.broadcast_to(dv, (NBLK, 112))], axis=1)

    return pl.pallas_call(
        body,
        grid=(GRID,),
        in_specs=[
            pl.BlockSpec((NBLK, 128), lambda n: (n, 0)),
            pl.BlockSpec((NBLK, 16), lambda n: (n, 0)),
        ],
        out_specs=pl.BlockSpec((NBLK, 128), lambda n: (n, 0)),
        out_shape=jax.ShapeDtypeStruct((NP, 128), jnp.float32),
    )(degf, x_pad)


def _tc_layer1(aggf, f0, w, b):
    def body(agg_ref, f0_ref, w_ref, b_ref, out_ref):
        dv = f0_ref[:, 16:17]
        p = dv * (agg_ref[:, 0:16] + agg_ref[:, 16:32] + f0_ref[:, 0:16])
        h = jnp.dot(p, w_ref[...], preferred_element_type=jnp.float32)
        h = jnp.maximum(h + b_ref[...], 0.0)
        g = dv * h
        out_ref[...] = jnp.concatenate(
            [g, jnp.broadcast_to(dv, (NBLK, 64))], axis=1)

    return pl.pallas_call(
        body,
        grid=(GRID,),
        in_specs=[
            pl.BlockSpec((NBLK, 128), lambda n: (n, 0)),
            pl.BlockSpec((NBLK, 128), lambda n: (n, 0)),
            pl.BlockSpec((16, 64), lambda n: (0, 0)),
            pl.BlockSpec((1, 64), lambda n: (0, 0)),
        ],
        out_specs=pl.BlockSpec((NBLK, 128), lambda n: (n, 0)),
        out_shape=jax.ShapeDtypeStruct((NP, 128), jnp.float32),
    )(aggf, f0, w, b)


def _tc_layer2(aggf, f1, w, b):
    def body(agg_ref, f1_ref, w_ref, b_ref, out_ref):
        dv = f1_ref[:, 64:65]
        p = dv * (agg_ref[:, 0:64] + f1_ref[:, 0:64])
        h = jnp.dot(p, w_ref[...], preferred_element_type=jnp.float32)
        h = jnp.maximum(h + b_ref[...], 0.0)
        out_ref[...] = dv * h

    return pl.pallas_call(
        body,
        grid=(GRID,),
        in_specs=[
            pl.BlockSpec((NBLK, 128), lambda n: (n, 0)),
            pl.BlockSpec((NBLK, 128), lambda n: (n, 0)),
            pl.BlockSpec((64, 128), lambda n: (0, 0)),
            pl.BlockSpec((1, 128), lambda n: (0, 0)),
        ],
        out_specs=pl.BlockSpec((NBLK, 128), lambda n: (n, 0)),
        out_shape=jax.ShapeDtypeStruct((NP, 128), jnp.float32),
    )(aggf, f1, w, b)


def _tc_layer3_head(f1, aggf, f2, w3, b3, fw1, fb1, fw2, fb2):
    def body(f1_ref, agg_ref, f2_ref, w3_ref, b3_ref, fw1_ref, fb1_ref,
             fw2_ref, fb2_ref, out_ref):
        dv = f1_ref[:, 64:65]
        p = dv * (agg_ref[...] + f2_ref[...])
        h = jnp.dot(p, w3_ref[...], preferred_element_type=jnp.float32)
        h = jnp.maximum(h + b3_ref[...], 0.0)
        z = jnp.dot(h, fw1_ref[...], preferred_element_type=jnp.float32)
        z = jnp.maximum(z + fb1_ref[...], 0.0)
        o = jnp.dot(z, fw2_ref[...], preferred_element_type=jnp.float32)
        out_ref[...] = o + fb2_ref[...]

    return pl.pallas_call(
        body,
        grid=(GRID,),
        in_specs=[
            pl.BlockSpec((NBLK, 128), lambda n: (n, 0)),
            pl.BlockSpec((NBLK, 128), lambda n: (n, 0)),
            pl.BlockSpec((NBLK, 128), lambda n: (n, 0)),
            pl.BlockSpec((128, 128), lambda n: (0, 0)),
            pl.BlockSpec((1, 128), lambda n: (0, 0)),
            pl.BlockSpec((128, 64), lambda n: (0, 0)),
            pl.BlockSpec((1, 64), lambda n: (0, 0)),
            pl.BlockSpec((64, 8), lambda n: (0, 0)),
            pl.BlockSpec((1, 8), lambda n: (0, 0)),
        ],
        out_specs=pl.BlockSpec((NBLK, 8), lambda n: (n, 0)),
        out_shape=jax.ShapeDtypeStruct((NP, 8), jnp.float32),
    )(f1, aggf, f2, w3, b3, fw1, fb1, fw2, fb2)


def kernel(x, edge_index, W1, b1, W2, b2, W3, b3, fW1, fb1, fW2, fb2):
    epk = (edge_index[1] << 16) | edge_index[0]
    fill = jnp.full((EP - E,), ((NP - 1) << 16) | (NP - 1), jnp.int32)
    epk2 = jnp.concatenate([epk, fill]).reshape(EP // B, B)
    x_pad = jnp.zeros((NP, 16), jnp.float32).at[:N, :3].set(x)
    w1p = jnp.zeros((16, 64), jnp.float32).at[:3].set(W1)
    fw2p = jnp.zeros((64, 8), jnp.float32).at[:, :2].set(fW2)
    fb2p = jnp.zeros((8,), jnp.float32).at[:2].set(fb2)

    degf = _sc_deg(epk2)
    f0 = _tc_prep(degf, x_pad)

    agg0 = _sc_agg1(epk2, f0.reshape(NP * 8, 16))
    f1 = _tc_layer1(agg0, f0, w1p, b1.reshape(1, 64))

    agg1 = _sc_agg2(epk2, f1.reshape(NP * 4, 32))
    f2 = _tc_layer2(agg1, f1, W2, b2.reshape(1, 128))

    agg2 = _sc_agg4(epk2, f2.reshape(NP * 4, 32))
    outp = _tc_layer3_head(f1, agg2, f2,
                           W3, b3.reshape(1, 128),
                           fW1, fb1.reshape(1, 64),
                           fw2p, fb2p.reshape(1, 8))
    return outp[:N, :2]
